# baseline (speedup 1.0000x reference)
import jax
import jax.numpy as jnp
from jax import lax
from jax.experimental import pallas as pl
from jax.experimental.pallas import tpu as pltpu
from jax.experimental.pallas import tpu_sc as plsc

N_NODES_C = 10000
N_EDGES_C = 320000
D = 128

NC = 2
NS = 16
NW = NC * NS

CH = 128
NCHUNKS = N_EDGES_C // CH
BASE = NCHUNKS // NW
EXTRA = NCHUNKS - BASE * NW
NBUF = 3
N_TURNS = (BASE - NBUF) // NBUF

ROWS_MAIN = 624
ROWS_LAST = 640


def _sc_body(feat_hbm, e_hbm, out_hbm,
             ibuf, rows0, rows1, rows2, acc,
             gsem0, gsem1, gsem2, isem0, isem1, isem2):
    c = lax.axis_index("c")
    s = lax.axis_index("s")
    wid = s * NC + c
    start = wid * BASE + jnp.minimum(wid, EXTRA)
    has_extra = wid < EXTRA

    zeros16 = jnp.zeros((16,), jnp.float32)

    def zrow(i, _):
        for j in range(D // 16):
            rows0[i, pl.ds(j * 16, 16)] = zeros16
        return 0

    lax.fori_loop(0, CH, zrow, 0, unroll=False)

    def zcopy(j, _):
        pltpu.sync_copy(rows0, acc.at[pl.ds(s * ROWS_MAIN + j * CH, CH)])
        return 0

    lax.fori_loop(0, ROWS_MAIN // CH, zcopy, 0, unroll=False)
    zbase = (ROWS_MAIN // CH) * CH

    @pl.when(s < NS - 1)
    def _():
        pltpu.sync_copy(rows0.at[pl.ds(0, ROWS_MAIN - zbase)],
                        acc.at[pl.ds(s * ROWS_MAIN + zbase, ROWS_MAIN - zbase)])

    @pl.when(s == NS - 1)
    def _():
        pltpu.sync_copy(rows0,
                        acc.at[pl.ds((NS - 1) * ROWS_MAIN + zbase, CH)])

    bufs = (rows0, rows1, rows2)
    gsems = (gsem0, gsem1, gsem2)
    isems = (isem0, isem1, isem2)

    def eref(m):
        return e_hbm.at[:, pl.ds(m * CH, CH)]

    def idxload(m, j):
        pltpu.async_copy(eref(m), ibuf.at[j], isems[j])

    def idxwait(m, j):
        pltpu.make_async_copy(eref(m), ibuf.at[j], isems[j]).wait()

    def gather(j):
        pltpu.async_copy(feat_hbm.at[ibuf.at[j, 0]], bufs[j], gsems[j])

    def gatherwait(j):
        pltpu.make_async_copy(feat_hbm.at[ibuf.at[j, 0]], bufs[j], gsems[j]).wait()

    def scatter(j):
        pltpu.sync_copy(bufs[j], acc.at[ibuf.at[j, 1]], add=True)

    for j in range(NBUF):
        idxload(start + j, j)
    for j in range(NBUF):
        idxwait(start + j, j)
        gather(j)

    plsc.subcore_barrier()

    def body(g, _):
        lb = NBUF * g
        for j in range(NBUF):
            gatherwait(j)
            idxload(start + lb + j + NBUF, j)
            scatter(j)
            idxwait(start + lb + j + NBUF, j)
            gather(j)
        return 0

    lax.fori_loop(0, N_TURNS, body, 0, unroll=False)

    lt = BASE - NBUF

    gatherwait(0)

    @pl.when(has_extra)
    def _():
        idxload(start + BASE, 0)

    scatter(0)

    @pl.when(has_extra)
    def _():
        idxwait(start + BASE, 0)
        gather(0)

    for j in range(1, NBUF):
        gatherwait(j)
        scatter(j)

    @pl.when(has_extra)
    def _():
        gatherwait(0)
        scatter(0)

    plsc.subcore_barrier()

    @pl.when(s < NS - 1)
    def _():
        pltpu.sync_copy(acc.at[pl.ds(s * ROWS_MAIN, ROWS_MAIN)],
                        out_hbm.at[pl.ds(c * N_NODES_C + s * ROWS_MAIN, ROWS_MAIN)])

    @pl.when(s == NS - 1)
    def _():
        pltpu.sync_copy(
            acc.at[pl.ds((NS - 1) * ROWS_MAIN, ROWS_LAST)],
            out_hbm.at[pl.ds(c * N_NODES_C + (NS - 1) * ROWS_MAIN, ROWS_LAST)])


@jax.jit
def _sc_aggregate(feature, edge_index):
    mesh = plsc.VectorSubcoreMesh(core_axis_name="c", subcore_axis_name="s")
    f = pl.kernel(
        _sc_body,
        out_type=jax.ShapeDtypeStruct((NC * N_NODES_C, D), jnp.float32),
        mesh=mesh,
        scratch_types=(
            [pltpu.VMEM((NBUF, 2, CH), jnp.int32)]
            + [pltpu.VMEM((CH, D), jnp.float32)] * NBUF
            + [pltpu.VMEM_SHARED((N_NODES_C, D), jnp.float32)]
            + [pltpu.SemaphoreType.DMA] * (2 * NBUF)
        ),
    )
    return f(feature, edge_index)


def _tc_body(p0_ref, p1_ref, wt_ref, b_ref, o_ref):
    agg = p0_ref[...] + p1_ref[...]
    h = jnp.dot(agg, wt_ref[...], preferred_element_type=jnp.float32)
    o_ref[...] = jnp.maximum(h + b_ref[...], 0.0)


@jax.jit
def _tc_update(partials, Wt, b2):
    blk = 10000
    grid = N_NODES_C // blk
    return pl.pallas_call(
        _tc_body,
        grid=(grid,),
        in_specs=[
            pl.BlockSpec((blk, D), lambda i: (i, 0)),
            pl.BlockSpec((blk, D), lambda i: (i + grid, 0)),
            pl.BlockSpec((D, D), lambda i: (0, 0)),
            pl.BlockSpec((1, D), lambda i: (0, 0)),
        ],
        out_specs=pl.BlockSpec((blk, D), lambda i: (i, 0)),
        out_shape=jax.ShapeDtypeStruct((N_NODES_C, D), jnp.float32),
    )(partials, partials, Wt, b2)


def kernel(feature, edge_index, W, b):
    partials = _sc_aggregate(feature, edge_index)
    return _tc_update(partials, W.T, b.reshape(1, D))


# speedup vs baseline: 1.0041x; 1.0041x over previous
import jax
import jax.numpy as jnp
from jax import lax
from jax.experimental import pallas as pl
from jax.experimental.pallas import tpu as pltpu
from jax.experimental.pallas import tpu_sc as plsc

N_NODES_C = 10000
N_EDGES_C = 320000
D = 128

NC = 2
NS = 16
NW = NC * NS

CH = 128
NCHUNKS = N_EDGES_C // CH
BASE = NCHUNKS // NW
EXTRA = NCHUNKS - BASE * NW
NBUF = 3
N_TURNS = (BASE - NBUF) // NBUF

ROWS_MAIN = 624
ROWS_LAST = 640


def _sc_body(feat_hbm, e_hbm, out_hbm,
             ibuf, dbuf, rows0, rows1, rows2, acc,
             gsem0, gsem1, gsem2, isem0, isem1, isem2,
             bsem0, bsem1, bsem2):
    c = lax.axis_index("c")
    s = lax.axis_index("s")
    wid = s * NC + c
    start = wid * BASE + jnp.minimum(wid, EXTRA)
    has_extra = wid < EXTRA

    zeros16 = jnp.zeros((16,), jnp.float32)

    def zrow(i, _):
        for j in range(D // 16):
            rows0[i, pl.ds(j * 16, 16)] = zeros16
        return 0

    lax.fori_loop(0, CH, zrow, 0, unroll=False)

    def zcopy(j, _):
        pltpu.sync_copy(rows0, acc.at[pl.ds(s * ROWS_MAIN + j * CH, CH)])
        return 0

    lax.fori_loop(0, ROWS_MAIN // CH, zcopy, 0, unroll=False)
    zbase = (ROWS_MAIN // CH) * CH

    @pl.when(s < NS - 1)
    def _():
        pltpu.sync_copy(rows0.at[pl.ds(0, ROWS_MAIN - zbase)],
                        acc.at[pl.ds(s * ROWS_MAIN + zbase, ROWS_MAIN - zbase)])

    @pl.when(s == NS - 1)
    def _():
        pltpu.sync_copy(rows0,
                        acc.at[pl.ds((NS - 1) * ROWS_MAIN + zbase, CH)])

    bufs = (rows0, rows1, rows2)
    gsems = (gsem0, gsem1, gsem2)
    isems = (isem0, isem1, isem2)
    bsems = (bsem0, bsem1, bsem2)

    def sref(m):
        return e_hbm.at[0, pl.ds(m * CH, CH)]

    def bref(m):
        return e_hbm.at[:, pl.ds(m * CH, CH)]

    def idxload(m, j):
        pltpu.async_copy(sref(m), ibuf.at[j], isems[j])

    def idxwait(m, j):
        pltpu.make_async_copy(sref(m), ibuf.at[j], isems[j]).wait()

    def blockload(m, j):
        pltpu.async_copy(bref(m), dbuf.at[j], bsems[j])

    def blockwait(m, j):
        pltpu.make_async_copy(bref(m), dbuf.at[j], bsems[j]).wait()

    def gather(j):
        pltpu.async_copy(feat_hbm.at[ibuf.at[j]], bufs[j], gsems[j])

    def gatherwait(j):
        pltpu.make_async_copy(feat_hbm.at[ibuf.at[j]], bufs[j], gsems[j]).wait()

    def scatter(j):
        pltpu.sync_copy(bufs[j], acc.at[dbuf.at[j, 1]], add=True)

    for j in range(NBUF):
        idxload(start + j, j)
        blockload(start + j, j)
    for j in range(NBUF):
        idxwait(start + j, j)
        gather(j)

    plsc.subcore_barrier()

    def body(g, _):
        lb = NBUF * g
        for j in range(NBUF):
            c = start + lb + j
            gatherwait(j)
            idxload(c + NBUF, j)
            blockwait(c, j)
            scatter(j)
            blockload(c + NBUF, j)
            idxwait(c + NBUF, j)
            gather(j)
        return 0

    lax.fori_loop(0, N_TURNS, body, 0, unroll=False)

    tc0 = start + BASE - NBUF

    gatherwait(0)

    @pl.when(has_extra)
    def _():
        idxload(start + BASE, 0)

    blockwait(tc0, 0)
    scatter(0)

    @pl.when(has_extra)
    def _():
        blockload(start + BASE, 0)
        idxwait(start + BASE, 0)
        gather(0)

    for j in range(1, NBUF):
        gatherwait(j)
        blockwait(tc0 + j, j)
        scatter(j)

    @pl.when(has_extra)
    def _():
        gatherwait(0)
        blockwait(start + BASE, 0)
        scatter(0)

    plsc.subcore_barrier()

    @pl.when(s < NS - 1)
    def _():
        pltpu.sync_copy(acc.at[pl.ds(s * ROWS_MAIN, ROWS_MAIN)],
                        out_hbm.at[pl.ds(c * N_NODES_C + s * ROWS_MAIN, ROWS_MAIN)])

    @pl.when(s == NS - 1)
    def _():
        pltpu.sync_copy(
            acc.at[pl.ds((NS - 1) * ROWS_MAIN, ROWS_LAST)],
            out_hbm.at[pl.ds(c * N_NODES_C + (NS - 1) * ROWS_MAIN, ROWS_LAST)])


@jax.jit
def _sc_aggregate(feature, edge_index):
    mesh = plsc.VectorSubcoreMesh(core_axis_name="c", subcore_axis_name="s")
    f = pl.kernel(
        _sc_body,
        out_type=jax.ShapeDtypeStruct((NC * N_NODES_C, D), jnp.float32),
        mesh=mesh,
        scratch_types=(
            [pltpu.VMEM((NBUF, CH), jnp.int32),
             pltpu.VMEM((NBUF, 2, CH), jnp.int32)]
            + [pltpu.VMEM((CH, D), jnp.float32)] * NBUF
            + [pltpu.VMEM_SHARED((N_NODES_C, D), jnp.float32)]
            + [pltpu.SemaphoreType.DMA] * (3 * NBUF)
        ),
    )
    return f(feature, edge_index)


def _tc_body(p0_ref, p1_ref, wt_ref, b_ref, o_ref):
    agg = p0_ref[...] + p1_ref[...]
    h = jnp.dot(agg, wt_ref[...], preferred_element_type=jnp.float32)
    o_ref[...] = jnp.maximum(h + b_ref[...], 0.0)


@jax.jit
def _tc_update(partials, Wt, b2):
    blk = 5000
    grid = N_NODES_C // blk
    return pl.pallas_call(
        _tc_body,
        grid=(grid,),
        in_specs=[
            pl.BlockSpec((blk, D), lambda i: (i, 0)),
            pl.BlockSpec((blk, D), lambda i: (i + grid, 0)),
            pl.BlockSpec((D, D), lambda i: (0, 0)),
            pl.BlockSpec((1, D), lambda i: (0, 0)),
        ],
        out_specs=pl.BlockSpec((blk, D), lambda i: (i, 0)),
        out_shape=jax.ShapeDtypeStruct((N_NODES_C, D), jnp.float32),
    )(partials, partials, Wt, b2)


def kernel(feature, edge_index, W, b):
    partials = _sc_aggregate(feature, edge_index)
    return _tc_update(partials, W.T, b.reshape(1, D))
